# Initial kernel scaffold; baseline (speedup 1.0000x reference)
import jax
import jax.numpy as jnp
from jax.experimental import pallas as pl


def kernel(x, adj1, adj2, adj3, Wx_w, Wx_b, Wn_w, Wn_b, bn_gamma, bn_beta):
    raise NotImplementedError("write your pallas kernel here")


# speedup vs baseline: 182.9113x; 182.9113x over previous
import jax
import jax.numpy as jnp
from jax.experimental import pallas as pl


def _sage_kernel(x_ref, a1_ref, a2_ref, a3_ref, wxT_ref, wnT_ref,
                 bx_ref, bn_ref, g_ref, be_ref, out_ref):
    B, N, F = x_ref.shape
    C = out_ref.shape[2]
    n_max = jnp.float32(N - 1)

    row = jax.lax.broadcasted_iota(jnp.int32, (N, N), 0)
    col = jax.lax.broadcasted_iota(jnp.int32, (N, N), 1)
    not_eye = row != col

    masks = []
    for a_ref in (a1_ref, a2_ref, a3_ref):
        m = jnp.where((a_ref[...] > 0.0) & not_eye, jnp.float32(1.0),
                      jnp.float32(0.0))
        k = jnp.sum(m, axis=1, keepdims=True)
        masks.append((m, k))

    wxT = wxT_ref[...]
    wnT = wnT_ref[...]
    bx = bx_ref[...]
    bn = bn_ref[...]

    s = jnp.zeros((1, C), dtype=jnp.float32)
    s2 = jnp.zeros((1, C), dtype=jnp.float32)
    for b in range(B):
        xb = x_ref[b]
        h0 = jnp.dot(xb, wxT, preferred_element_type=jnp.float32) + bx
        y = jnp.dot(xb, wnT, preferred_element_type=jnp.float32)
        y_last = y[N - 1:N, :]
        hs = [h0]
        for m, k in masks:
            agg = jnp.dot(m, y, preferred_element_type=jnp.float32)
            agg = (agg + (n_max - k) * y_last) / n_max
            hs.append(agg + bn)
        h = jnp.concatenate(hs, axis=1)
        nrm = jnp.sqrt(jnp.sum(h * h, axis=1, keepdims=True))
        h = h / jnp.maximum(nrm, jnp.float32(1e-12))
        h = jnp.maximum(h, jnp.float32(0.0))
        out_ref[b] = h
        s = s + jnp.sum(h, axis=0, keepdims=True)
        s2 = s2 + jnp.sum(h * h, axis=0, keepdims=True)

    cnt = jnp.float32(B * N)
    mean = s / cnt
    var = s2 / cnt - mean * mean
    scale = g_ref[...] / jnp.sqrt(var + jnp.float32(1e-5))
    shift = be_ref[...] - mean * scale
    for b in range(B):
        out_ref[b] = out_ref[b] * scale + shift


def kernel(x, adj1, adj2, adj3, Wx_w, Wx_b, Wn_w, Wn_b, bn_gamma, bn_beta):
    B, N, F = x.shape
    O = Wx_w.shape[0]
    C = 4 * O
    out = pl.pallas_call(
        _sage_kernel,
        out_shape=jax.ShapeDtypeStruct((B, N, C), jnp.float32),
    )(x, adj1, adj2, adj3,
      Wx_w.T, Wn_w.T,
      Wx_b.reshape(1, O), Wn_b.reshape(1, O),
      bn_gamma.reshape(1, C), bn_beta.reshape(1, C))
    return out


# speedup vs baseline: 321.9954x; 1.7604x over previous
import jax
import jax.numpy as jnp
from jax.experimental import pallas as pl

_CONTRACT_RHS1 = (((1,), (1,)), ((), ()))


def _sage_kernel(x_ref, a1_ref, a2_ref, a3_ref, wx_ref, wn_ref,
                 bx_ref, bn_ref, g_ref, be_ref, out_ref):
    B, N, F = x_ref.shape
    C = out_ref.shape[2]
    O = C // 4
    n_max = jnp.float32(N - 1)

    row = jax.lax.broadcasted_iota(jnp.int32, (N, N), 0)
    col = jax.lax.broadcasted_iota(jnp.int32, (N, N), 1)
    not_eye = row != col

    ms = [jnp.where((a_ref[...] > 0.0) & not_eye, jnp.float32(1.0),
                    jnp.float32(0.0))
          for a_ref in (a1_ref, a2_ref, a3_ref)]
    m_all = jnp.concatenate(ms, axis=0)
    corr = n_max - jnp.sum(m_all, axis=1, keepdims=True)

    x_all = x_ref[...].reshape(B * N, F)
    h0_all = jax.lax.dot_general(x_all, wx_ref[...], _CONTRACT_RHS1,
                                 preferred_element_type=jnp.float32)
    h0_all = h0_all + bx_ref[...]
    y_all = jax.lax.dot_general(x_all, wn_ref[...], _CONTRACT_RHS1,
                                preferred_element_type=jnp.float32)

    bn = bn_ref[...]
    s = jnp.zeros((1, C), dtype=jnp.float32)
    s2 = jnp.zeros((1, C), dtype=jnp.float32)
    for b in range(B):
        y_b = y_all[b * N:(b + 1) * N, :]
        y_last = y_all[b * N + N - 1:b * N + N, :]
        agg = jnp.dot(m_all, y_b, preferred_element_type=jnp.float32)
        agg = (agg + corr * y_last) / n_max + bn
        h = jnp.concatenate(
            [h0_all[b * N:(b + 1) * N, :],
             agg[0:N, :], agg[N:2 * N, :], agg[2 * N:3 * N, :]], axis=1)
        nrm = jnp.sqrt(jnp.sum(h * h, axis=1, keepdims=True))
        h = h / jnp.maximum(nrm, jnp.float32(1e-12))
        h = jnp.maximum(h, jnp.float32(0.0))
        out_ref[b] = h
        s = s + jnp.sum(h, axis=0, keepdims=True)
        s2 = s2 + jnp.sum(h * h, axis=0, keepdims=True)

    cnt = jnp.float32(B * N)
    mean = s / cnt
    var = s2 / cnt - mean * mean
    scale = g_ref[...] / jnp.sqrt(var + jnp.float32(1e-5))
    shift = be_ref[...] - mean * scale
    for b in range(B):
        out_ref[b] = out_ref[b] * scale + shift


def kernel(x, adj1, adj2, adj3, Wx_w, Wx_b, Wn_w, Wn_b, bn_gamma, bn_beta):
    B, N, F = x.shape
    O = Wx_w.shape[0]
    C = 4 * O
    out = pl.pallas_call(
        _sage_kernel,
        out_shape=jax.ShapeDtypeStruct((B, N, C), jnp.float32),
    )(x, adj1, adj2, adj3, Wx_w, Wn_w,
      Wx_b.reshape(1, O), Wn_b.reshape(1, O),
      bn_gamma.reshape(1, C), bn_beta.reshape(1, C))
    return out


# speedup vs baseline: 323.1888x; 1.0037x over previous
import jax
import jax.numpy as jnp
from jax.experimental import pallas as pl

_CONTRACT_RHS1 = (((1,), (1,)), ((), ()))


def _sage_kernel(x_ref, a1_ref, a2_ref, a3_ref, wx_ref, wn_ref,
                 bx_ref, bn_ref, g_ref, be_ref, out_ref):
    B, N, F = x_ref.shape
    C = out_ref.shape[2]
    O = C // 4
    n_max = jnp.float32(N - 1)

    row = jax.lax.broadcasted_iota(jnp.int32, (N, N), 0)
    col = jax.lax.broadcasted_iota(jnp.int32, (N, N), 1)
    not_eye = row != col

    inv_n = jnp.float32(1.0) / n_max
    ms = [jnp.where((a_ref[...] > 0.0) & not_eye, inv_n, jnp.float32(0.0))
          for a_ref in (a1_ref, a2_ref, a3_ref)]
    m_all = jnp.concatenate(ms, axis=0)
    corr = jnp.float32(1.0) - jnp.sum(m_all, axis=1, keepdims=True)

    x_all = x_ref[...].reshape(B * N, F)
    h0_all = jax.lax.dot_general(x_all, wx_ref[...], _CONTRACT_RHS1,
                                 preferred_element_type=jnp.float32)
    h0_all = h0_all + bx_ref[...]
    y_all = jax.lax.dot_general(x_all, wn_ref[...], _CONTRACT_RHS1,
                                preferred_element_type=jnp.float32)

    bn = bn_ref[...]
    s = jnp.zeros((1, C), dtype=jnp.float32)
    s2 = jnp.zeros((1, C), dtype=jnp.float32)
    for b in range(B):
        y_b = y_all[b * N:(b + 1) * N, :]
        y_last = y_all[b * N + N - 1:b * N + N, :]
        agg = jnp.dot(m_all, y_b, preferred_element_type=jnp.float32)
        agg = agg + corr * y_last + bn
        h = jnp.concatenate(
            [h0_all[b * N:(b + 1) * N, :],
             agg[0:N, :], agg[N:2 * N, :], agg[2 * N:3 * N, :]], axis=1)
        nrm = jnp.sqrt(jnp.sum(h * h, axis=1, keepdims=True))
        h = h / jnp.maximum(nrm, jnp.float32(1e-12))
        h = jnp.maximum(h, jnp.float32(0.0))
        out_ref[b] = h
        s = s + jnp.sum(h, axis=0, keepdims=True)
        s2 = s2 + jnp.sum(h * h, axis=0, keepdims=True)

    cnt = jnp.float32(B * N)
    mean = s / cnt
    var = s2 / cnt - mean * mean
    scale = g_ref[...] / jnp.sqrt(var + jnp.float32(1e-5))
    shift = be_ref[...] - mean * scale
    for b in range(B):
        out_ref[b] = out_ref[b] * scale + shift


def kernel(x, adj1, adj2, adj3, Wx_w, Wx_b, Wn_w, Wn_b, bn_gamma, bn_beta):
    B, N, F = x.shape
    O = Wx_w.shape[0]
    C = 4 * O
    out = pl.pallas_call(
        _sage_kernel,
        out_shape=jax.ShapeDtypeStruct((B, N, C), jnp.float32),
    )(x, adj1, adj2, adj3, Wx_w, Wn_w,
      Wx_b.reshape(1, O), Wn_b.reshape(1, O),
      bn_gamma.reshape(1, C), bn_beta.reshape(1, C))
    return out
